# Initial kernel scaffold; baseline (speedup 1.0000x reference)
import functools

import jax
import jax.numpy as jnp
from jax import lax
from jax.experimental import pallas as pl
from jax.experimental.pallas import tpu as pltpu
from jax.experimental.pallas import tpu_sc as plsc

NUM_EMB = 1_000_000
D = 32
B = 16384 * 50
NC, NS = 2, 16
NW = NC * NS
SUB = 128
K = 8
C = SUB * K
CHUNKS = B // (NW * C)

_mesh = plsc.VectorSubcoreMesh(core_axis_name="c", subcore_axis_name="s")


@functools.partial(
    pl.kernel,
    mesh=_mesh,
    out_type=jax.ShapeDtypeStruct((B, D), jnp.float32),
    scratch_types=[
        pltpu.VMEM((K, SUB), jnp.int32),
        pltpu.VMEM((C, D), jnp.float32),
        pltpu.SemaphoreType.DMA,
    ],
)
def _embed_gather(idx_hbm, table_hbm, out_hbm, idx_v, rows_v, sem):
    wid = lax.axis_index("s") * NC + lax.axis_index("c")

    def body(g, carry):
        row0 = (wid * CHUNKS + g) * C
        pltpu.sync_copy(idx_hbm.at[pl.ds(row0 // SUB, K)], idx_v)
        copies = []
        for j in range(K):
            copies.append(
                pltpu.async_copy(
                    table_hbm.at[idx_v.at[j]],
                    rows_v.at[pl.ds(j * SUB, SUB)],
                    sem,
                )
            )
        for cp in copies:
            cp.wait()
        pltpu.sync_copy(rows_v, out_hbm.at[pl.ds(row0, C)])
        return carry

    lax.fori_loop(0, CHUNKS, body, 0)


def kernel(token_ids, M_embed):
    flat = token_ids.reshape(-1).astype(jnp.int32)
    idx2d = flat.reshape(B // SUB, SUB)
    out = _embed_gather(idx2d, M_embed)
    return out.reshape(*token_ids.shape, D)


# speedup vs baseline: 1.0937x; 1.0937x over previous
import functools

import jax
import jax.numpy as jnp
from jax import lax
from jax.experimental import pallas as pl
from jax.experimental.pallas import tpu as pltpu
from jax.experimental.pallas import tpu_sc as plsc

NUM_EMB = 1_000_000
D = 32
B = 16384 * 50
NC, NS = 2, 16
NW = NC * NS
SUB = 128
K = 8
C = SUB * K
CHUNKS = B // (NW * C)

_mesh = plsc.VectorSubcoreMesh(core_axis_name="c", subcore_axis_name="s")


@functools.partial(
    pl.kernel,
    mesh=_mesh,
    compiler_params=pltpu.CompilerParams(use_tc_tiling_on_sc=False),
    out_type=jax.ShapeDtypeStruct((B, D), jnp.float32),
    scratch_types=[
        pltpu.VMEM((K, SUB), jnp.int32),
        pltpu.VMEM((C, D), jnp.float32),
        pltpu.SemaphoreType.DMA,
    ],
)
def _embed_gather(idx_hbm, table_hbm, out_hbm, idx_v, rows_v, sem):
    wid = lax.axis_index("s") * NC + lax.axis_index("c")

    def body(g, carry):
        row0 = pl.multiple_of((wid * CHUNKS + g) * C, C)
        pltpu.sync_copy(idx_hbm.at[pl.ds(pl.multiple_of(row0 // SUB, K), K)], idx_v)
        copies = []
        for j in range(K):
            copies.append(
                pltpu.async_copy(
                    table_hbm.at[idx_v.at[j]],
                    rows_v.at[pl.ds(j * SUB, SUB)],
                    sem,
                )
            )
        for cp in copies:
            cp.wait()
        pltpu.sync_copy(rows_v, out_hbm.at[pl.ds(row0, C)])
        return carry

    lax.fori_loop(0, CHUNKS, body, 0)


def kernel(token_ids, M_embed):
    flat = token_ids.reshape(-1).astype(jnp.int32)
    idx2d = flat.reshape(B // SUB, SUB)
    out = _embed_gather(idx2d, M_embed)
    return out.reshape(*token_ids.shape, D)


# speedup vs baseline: 1.1095x; 1.0145x over previous
import functools

import jax
import jax.numpy as jnp
from jax import lax
from jax.experimental import pallas as pl
from jax.experimental.pallas import tpu as pltpu
from jax.experimental.pallas import tpu_sc as plsc

NUM_EMB = 1_000_000
D = 32
B = 16384 * 50
NC, NS = 2, 16
NW = NC * NS
SUB = 128
K = 10
C = SUB * K
CHUNKS = B // (NW * C)

_mesh = plsc.VectorSubcoreMesh(core_axis_name="c", subcore_axis_name="s")


@functools.partial(
    pl.kernel,
    mesh=_mesh,
    compiler_params=pltpu.CompilerParams(use_tc_tiling_on_sc=False),
    out_type=jax.ShapeDtypeStruct((B, D), jnp.float32),
    scratch_types=[
        pltpu.VMEM((2, K, SUB), jnp.int32),
        pltpu.VMEM((2, C, D), jnp.float32),
        pltpu.SemaphoreType.DMA,
        pltpu.SemaphoreType.DMA,
        pltpu.SemaphoreType.DMA,
        pltpu.SemaphoreType.DMA,
        pltpu.SemaphoreType.DMA,
        pltpu.SemaphoreType.DMA,
    ],
)
def _embed_gather(idx_hbm, table_hbm, out_hbm, idx_v, rows_v,
                  sem_i0, sem_i1, sem_g0, sem_g1, sem_o0, sem_o1):
    wid = lax.axis_index("s") * NC + lax.axis_index("c")
    sem_i = (sem_i0, sem_i1)
    sem_g = (sem_g0, sem_g1)
    sem_o = (sem_o0, sem_o1)

    def row0(g):
        return pl.multiple_of((wid * CHUNKS + g) * C, C)

    def fire_idx_load(g, b):
        pltpu.async_copy(
            idx_hbm.at[pl.ds(pl.multiple_of(row0(g) // SUB, K), K)],
            idx_v.at[b], sem_i[b])

    def wait_idx(b):
        pltpu.make_async_copy(idx_hbm.at[pl.ds(0, K)], idx_v.at[b],
                              sem_i[b]).wait()

    def fire_gathers(b):
        for j in range(K):
            pltpu.async_copy(table_hbm.at[idx_v.at[b, j]],
                             rows_v.at[b, pl.ds(j * SUB, SUB)], sem_g[b])

    def wait_gathers(b):
        pltpu.make_async_copy(table_hbm.at[pl.ds(0, C)], rows_v.at[b],
                              sem_g[b]).wait()

    def fire_writeback(g, b):
        pltpu.async_copy(rows_v.at[b], out_hbm.at[pl.ds(row0(g), C)],
                         sem_o[b])

    def wait_writeback(b):
        pltpu.make_async_copy(rows_v.at[b], out_hbm.at[pl.ds(0, C)],
                              sem_o[b]).wait()

    def half(g, b):
        bo = 1 - b

        @pl.when(g >= 1)
        def _():
            wait_gathers(bo)
            fire_writeback(g - 1, bo)

        @pl.when(g + 1 <= CHUNKS - 1)
        def _():
            fire_idx_load(g + 1, bo)

        wait_idx(b)

        @pl.when(g >= 2)
        def _():
            wait_writeback(b)

        fire_gathers(b)

    fire_idx_load(0, 0)

    def body(p, carry):
        g = p * 2
        half(g, 0)
        half(g + 1, 1)
        return carry

    lax.fori_loop(0, CHUNKS // 2, body, 0)

    wait_gathers(1)
    fire_writeback(CHUNKS - 1, 1)
    wait_writeback(0)
    wait_writeback(1)


def kernel(token_ids, M_embed):
    flat = token_ids.reshape(-1).astype(jnp.int32)
    idx2d = flat.reshape(B // SUB, SUB)
    out = _embed_gather(idx2d, M_embed)
    return out.reshape(*token_ids.shape, D)


# speedup vs baseline: 1.5465x; 1.3939x over previous
import functools

import jax
import jax.numpy as jnp
from jax import lax
from jax.experimental import pallas as pl
from jax.experimental.pallas import tpu as pltpu
from jax.experimental.pallas import tpu_sc as plsc

R = 16384
S = 50
D = 32
NC, NS = 2, 16
NW = NC * NS
NKT = R // 128
KT_W = NKT // NW
FULL_CT = 6
TP_ROWS = 250000

_mesh = plsc.VectorSubcoreMesh(core_axis_name="c", subcore_axis_name="s")


@functools.partial(
    pl.kernel,
    mesh=_mesh,
    compiler_params=pltpu.CompilerParams(use_tc_tiling_on_sc=True,
                                         needs_layout_passes=False),
    out_type=jax.ShapeDtypeStruct((S, D, R), jnp.float32),
    scratch_types=[
        pltpu.VMEM((8, 128), jnp.int32),
        pltpu.VMEM((2, 128), jnp.int32),
        pltpu.VMEM((2, 128), jnp.int32),
        pltpu.VMEM((2, 128, 128), jnp.float32),
        pltpu.VMEM((2, 32, 128), jnp.float32),
        pltpu.SemaphoreType.DMA,
        pltpu.SemaphoreType.DMA,
        pltpu.SemaphoreType.DMA,
        pltpu.SemaphoreType.DMA,
    ],
)
def _embed_native(idxT, tableP, outT, idxt_v, q_v, o_v, rows_v, out_v,
                  sg0, sg1, so0, so1):
    wid = lax.axis_index("s") * NC + lax.axis_index("c")
    k0 = wid * KT_W
    iota = lax.iota(jnp.int32, 16)
    jvs = [iota + (16 * v) for v in range(8)]
    sg = (sg0, sg1)
    so = (so0, so1)

    def compute_qo(rr, b):
        for v in range(8):
            x = idxt_v[rr, pl.ds(16 * v, 16)]
            q_v[b, pl.ds(16 * v, 16)] = lax.shift_right_logical(x, 2)
            o_v[b, pl.ds(16 * v, 16)] = lax.shift_left(x & 3, 5)

    def fire_gather(b):
        return pltpu.async_copy(tableP.at[q_v.at[b]], rows_v.at[b], sg[b])

    def transpose_block(b):
        rows = rows_v.at[b]
        ovs = [o_v[b, pl.ds(16 * v, 16)] for v in range(8)]

        def dbody(d, carry):
            for v in range(8):
                x = plsc.load_gather(rows, [jvs[v], ovs[v] + d])
                out_v[b, d, pl.ds(16 * v, 16)] = x
            return carry

        lax.fori_loop(0, D, dbody, 0)

    def fire_out(b, c, col):
        return pltpu.async_copy(
            out_v.at[b], outT.at[c, :, pl.ds(col, 128)], so[b])

    def wait_out(b):
        pltpu.make_async_copy(
            out_v.at[b], outT.at[0, :, pl.ds(0, 128)], so[b]).wait()

    def ubody(u, carry):
        ct = u // KT_W
        kk = u - ct * KT_W
        c0 = pl.multiple_of(ct * 8, 8)
        col = pl.multiple_of((k0 + kk) * 128, 128)
        pltpu.sync_copy(idxT.at[pl.ds(c0, 8), pl.ds(col, 128)], idxt_v)

        compute_qo(0, 0)
        g = {0: fire_gather(0)}
        for r in range(8):
            b = r % 2
            if r + 1 < 8:
                nb = (r + 1) % 2
                compute_qo(r + 1, nb)
                g[r + 1] = fire_gather(nb)
            g[r].wait()
            @pl.when(jnp.logical_or(u > 0, r >= 2))
            def _():
                wait_out(b)
            transpose_block(b)
            fire_out(b, c0 + r, col)
        return carry

    lax.fori_loop(0, FULL_CT * KT_W, ubody, 0)
    wait_out(0)
    wait_out(1)

    for kk in range(KT_W):
        colk = pl.multiple_of((k0 + kk) * 128, 128)
        pltpu.sync_copy(idxT.at[pl.ds(48, 2), pl.ds(colk, 128)],
                        idxt_v.at[pl.ds(0, 2)])
        for r in range(2):
            compute_qo(r, 0)
            fire_gather(0).wait()
            transpose_block(0)
            pltpu.async_copy(
                out_v.at[0], outT.at[48 + r, :, pl.ds(colk, 128)],
                so0).wait()


def kernel(token_ids, M_embed):
    idxT = jnp.transpose(token_ids).astype(jnp.int32)
    tableP = M_embed.reshape(TP_ROWS, 128)
    out = _embed_native(idxT, tableP)
    return jnp.transpose(out, (2, 0, 1))


# speedup vs baseline: 2.0242x; 1.3089x over previous
import functools

import jax
import jax.numpy as jnp
from jax import lax
from jax.experimental import pallas as pl
from jax.experimental.pallas import tpu as pltpu
from jax.experimental.pallas import tpu_sc as plsc

R = 16384
S = 50
D = 32
NC, NS = 2, 16
NW = NC * NS
NKT = R // 128
KT_W = NKT // NW
FULL_CT = 6
TP_ROWS = 250000

_mesh = plsc.VectorSubcoreMesh(core_axis_name="c", subcore_axis_name="s")


@functools.partial(
    pl.kernel,
    mesh=_mesh,
    compiler_params=pltpu.CompilerParams(use_tc_tiling_on_sc=True,
                                         needs_layout_passes=False),
    out_type=jax.ShapeDtypeStruct((S, D, R), jnp.float32),
    scratch_types=[
        pltpu.VMEM((8, 128), jnp.int32),
        pltpu.VMEM((2, 128), jnp.int32),
        pltpu.VMEM((2, 128), jnp.int32),
        pltpu.VMEM((2, 128, 128), jnp.float32),
        pltpu.VMEM((2, 32, 128), jnp.float32),
        pltpu.SemaphoreType.DMA,
        pltpu.SemaphoreType.DMA,
        pltpu.SemaphoreType.DMA,
        pltpu.SemaphoreType.DMA,
    ],
)
def _embed_native(idxT, tableP, outT, idxt_v, q_v, o_v, rows_v, out_v,
                  sg0, sg1, so0, so1):
    wid = lax.axis_index("s") * NC + lax.axis_index("c")
    k0 = wid * KT_W
    iota = lax.iota(jnp.int32, 16)
    jvs = [iota + (16 * v) for v in range(8)]
    sg = (sg0, sg1)
    so = (so0, so1)

    def compute_qo(rr, b):
        for v in range(8):
            x = idxt_v[rr, pl.ds(16 * v, 16)]
            q_v[b, pl.ds(16 * v, 16)] = lax.shift_right_logical(x, 2)
            o_v[b, pl.ds(16 * v, 16)] = lax.shift_left(x & 3, 5)

    def fire_gather(b):
        return pltpu.async_copy(tableP.at[q_v.at[b]], rows_v.at[b], sg[b])

    def transpose_block(b):
        rows = rows_v.at[b]
        ovs = [o_v[b, pl.ds(16 * v, 16)] for v in range(8)]

        @plsc.parallel_loop(0, D, unroll=8)
        def dbody(d):
            for v in range(8):
                x = plsc.load_gather(rows, [jvs[v], ovs[v] + d])
                out_v[b, d, pl.ds(16 * v, 16)] = x

    def fire_out(b, c, col):
        return pltpu.async_copy(
            out_v.at[b], outT.at[c, :, pl.ds(col, 128)], so[b])

    def wait_out(b):
        pltpu.make_async_copy(
            out_v.at[b], outT.at[0, :, pl.ds(0, 128)], so[b]).wait()

    def ubody(u, carry):
        ct = u // KT_W
        kk = u - ct * KT_W
        c0 = pl.multiple_of(ct * 8, 8)
        col = pl.multiple_of((k0 + kk) * 128, 128)
        pltpu.sync_copy(idxT.at[pl.ds(c0, 8), pl.ds(col, 128)], idxt_v)

        compute_qo(0, 0)
        g = {0: fire_gather(0)}
        for r in range(8):
            b = r % 2
            if r + 1 < 8:
                nb = (r + 1) % 2
                compute_qo(r + 1, nb)
                g[r + 1] = fire_gather(nb)
            g[r].wait()
            @pl.when(jnp.logical_or(u > 0, r >= 2))
            def _():
                wait_out(b)
            transpose_block(b)
            fire_out(b, c0 + r, col)
        return carry

    lax.fori_loop(0, FULL_CT * KT_W, ubody, 0)
    wait_out(0)
    wait_out(1)

    for kk in range(KT_W):
        colk = pl.multiple_of((k0 + kk) * 128, 128)
        pltpu.sync_copy(idxT.at[pl.ds(48, 2), pl.ds(colk, 128)],
                        idxt_v.at[pl.ds(0, 2)])
        for r in range(2):
            compute_qo(r, 0)
            fire_gather(0).wait()
            transpose_block(0)
            pltpu.async_copy(
                out_v.at[0], outT.at[48 + r, :, pl.ds(colk, 128)],
                so0).wait()


def kernel(token_ids, M_embed):
    idxT = jnp.transpose(token_ids).astype(jnp.int32)
    tableP = M_embed.reshape(TP_ROWS, 128)
    out = _embed_native(idxT, tableP)
    return jnp.transpose(out, (2, 0, 1))


# speedup vs baseline: 2.2025x; 1.0881x over previous
import functools

import jax
import jax.numpy as jnp
from jax import lax
from jax.experimental import pallas as pl
from jax.experimental.pallas import tpu as pltpu
from jax.experimental.pallas import tpu_sc as plsc

R = 16384
S = 50
D = 32
NC, NS = 2, 16
NW = NC * NS
NKT = R // 128
KT_W = NKT // NW
FULL_CT = 6
TP_ROWS = 250000

_mesh = plsc.VectorSubcoreMesh(core_axis_name="c", subcore_axis_name="s")

NT = 7812
NT_W = 246
PART_COL = NT * 128


@functools.partial(
    pl.kernel,
    mesh=_mesh,
    compiler_params=pltpu.CompilerParams(use_tc_tiling_on_sc=True,
                                         needs_layout_passes=False),
    out_type=jax.ShapeDtypeStruct((TP_ROWS, 128), jnp.float32),
    scratch_types=[
        pltpu.VMEM((2, 32, 128), jnp.float32),
        pltpu.VMEM((2, 32, 128), jnp.float32),
        pltpu.SemaphoreType.DMA,
        pltpu.SemaphoreType.DMA,
        pltpu.SemaphoreType.DMA,
        pltpu.SemaphoreType.DMA,
    ],
)
def _pack_table(tableM, tailP, tableP, in_v, tp_v, si0, si1, so0, so1):
    wid = lax.axis_index("s") * NC + lax.axis_index("c")
    si = (si0, si1)
    so = (so0, so1)
    iota = lax.iota(jnp.int32, 16)
    izero = iota & 0
    dvec = (iota, iota + 16)

    def fire_in(j, b):
        col = pl.multiple_of((wid + j * NW) * 128, 128)
        return pltpu.async_copy(tableM.at[:, pl.ds(col, 128)], in_v.at[b],
                                si[b])

    def wait_in(b):
        pltpu.make_async_copy(tableM.at[:, pl.ds(0, 128)], in_v.at[b],
                              si[b]).wait()

    def wait_out(b):
        pltpu.make_async_copy(tp_v.at[b], tableP.at[pl.ds(0, 32)],
                              so[b]).wait()

    def half(p, jc, b):
        j = p * 2 + jc
        t = wid + j * NW

        @pl.when(t < NT)
        def _():
            wait_in(b)

            @pl.when(j >= 2)
            def _():
                wait_out(b)

            @plsc.parallel_loop(0, 32, unroll=8)
            def _trans(orow):
                for v in range(8):
                    cv = izero + (orow * 4 + v // 2)
                    x = plsc.load_gather(in_v.at[b], [dvec[v % 2], cv])
                    tp_v[b, orow, pl.ds(16 * v, 16)] = x

            row0 = pl.multiple_of(t * 32, 32)
            pltpu.async_copy(tp_v.at[b], tableP.at[pl.ds(row0, 32)], so[b])

        @pl.when(wid + (j + 2) * NW < NT)
        def _():
            fire_in(j + 2, b)

    fire_in(0, 0)
    fire_in(1, 1)

    def body(p, carry):
        half(p, 0, 0)
        half(p, 1, 1)
        return carry

    lax.fori_loop(0, NT_W // 2, body, 0)

    wait_out(0)
    wait_out(1)

    @pl.when(wid == 0)
    def _():
        pltpu.sync_copy(tailP, in_v.at[0, pl.ds(0, 16)])
        pltpu.sync_copy(in_v.at[0, pl.ds(0, 16)],
                        tableP.at[pl.ds(TP_ROWS - 16, 16)])


@functools.partial(
    pl.kernel,
    mesh=_mesh,
    compiler_params=pltpu.CompilerParams(use_tc_tiling_on_sc=True,
                                         needs_layout_passes=False),
    out_type=jax.ShapeDtypeStruct((S, D, R), jnp.float32),
    scratch_types=[
        pltpu.VMEM((8, 128), jnp.int32),
        pltpu.VMEM((2, 128), jnp.int32),
        pltpu.VMEM((2, 128), jnp.int32),
        pltpu.VMEM((2, 128, 128), jnp.float32),
        pltpu.VMEM((2, 32, 128), jnp.float32),
        pltpu.SemaphoreType.DMA,
        pltpu.SemaphoreType.DMA,
        pltpu.SemaphoreType.DMA,
        pltpu.SemaphoreType.DMA,
    ],
)
def _embed_native(idxT, tableP, outT, idxt_v, q_v, o_v, rows_v, out_v,
                  sg0, sg1, so0, so1):
    wid = lax.axis_index("s") * NC + lax.axis_index("c")
    k0 = wid * KT_W
    iota = lax.iota(jnp.int32, 16)
    jvs = [iota + (16 * v) for v in range(8)]
    sg = (sg0, sg1)
    so = (so0, so1)

    def compute_qo(rr, b):
        for v in range(8):
            x = idxt_v[rr, pl.ds(16 * v, 16)]
            q_v[b, pl.ds(16 * v, 16)] = lax.shift_right_logical(x, 2)
            o_v[b, pl.ds(16 * v, 16)] = lax.shift_left(x & 3, 5)

    def fire_gather(b):
        return pltpu.async_copy(tableP.at[q_v.at[b]], rows_v.at[b], sg[b])

    def transpose_block(b):
        rows = rows_v.at[b]
        ovs = [o_v[b, pl.ds(16 * v, 16)] for v in range(8)]

        @plsc.parallel_loop(0, D, unroll=8)
        def dbody(d):
            for v in range(8):
                x = plsc.load_gather(rows, [jvs[v], ovs[v] + d])
                out_v[b, d, pl.ds(16 * v, 16)] = x

    def fire_out(b, c, col):
        return pltpu.async_copy(
            out_v.at[b], outT.at[c, :, pl.ds(col, 128)], so[b])

    def wait_out(b):
        pltpu.make_async_copy(
            out_v.at[b], outT.at[0, :, pl.ds(0, 128)], so[b]).wait()

    def ubody(u, carry):
        ct = u // KT_W
        kk = u - ct * KT_W
        c0 = pl.multiple_of(ct * 8, 8)
        col = pl.multiple_of((k0 + kk) * 128, 128)
        pltpu.sync_copy(idxT.at[pl.ds(c0, 8), pl.ds(col, 128)], idxt_v)

        compute_qo(0, 0)
        g = {0: fire_gather(0)}
        for r in range(8):
            b = r % 2
            if r + 1 < 8:
                nb = (r + 1) % 2
                compute_qo(r + 1, nb)
                g[r + 1] = fire_gather(nb)
            g[r].wait()
            @pl.when(jnp.logical_or(u > 0, r >= 2))
            def _():
                wait_out(b)
            transpose_block(b)
            fire_out(b, c0 + r, col)
        return carry

    lax.fori_loop(0, FULL_CT * KT_W, ubody, 0)
    wait_out(0)
    wait_out(1)

    for kk in range(KT_W):
        colk = pl.multiple_of((k0 + kk) * 128, 128)
        pltpu.sync_copy(idxT.at[pl.ds(48, 2), pl.ds(colk, 128)],
                        idxt_v.at[pl.ds(0, 2)])
        for r in range(2):
            compute_qo(r, 0)
            fire_gather(0).wait()
            transpose_block(0)
            pltpu.async_copy(
                out_v.at[0], outT.at[48 + r, :, pl.ds(colk, 128)],
                so0).wait()


def kernel(token_ids, M_embed):
    idxT = jnp.transpose(token_ids).astype(jnp.int32)
    tableM = jnp.transpose(M_embed)
    tailP = M_embed[PART_COL:].reshape(16, 128)
    tableP = _pack_table(tableM, tailP)
    out = _embed_native(idxT, tableP)
    return jnp.transpose(out, (2, 0, 1))
